# initial kernel scaffold (unmeasured)
import jax
import jax.numpy as jnp
from jax import lax
from jax.experimental import pallas as pl
from jax.experimental.pallas import tpu as pltpu

T = 1024
V = 32768
V_HALF = V // 2
BM = 128


def _exchange_body(l_ref, out_ref, send_sem, recv_sem):
    my_x = lax.axis_index("x")
    my_y = lax.axis_index("y")
    my_z = lax.axis_index("z")
    nbr = (1 - my_x, my_y, my_z)

    barrier_sem = pltpu.get_barrier_semaphore()
    pl.semaphore_signal(
        barrier_sem, inc=1, device_id=nbr, device_id_type=pl.DeviceIdType.MESH
    )
    pl.semaphore_wait(barrier_sem, 1)

    rdma = pltpu.make_async_remote_copy(
        src_ref=l_ref,
        dst_ref=out_ref,
        send_sem=send_sem,
        recv_sem=recv_sem,
        device_id=nbr,
        device_id_type=pl.DeviceIdType.MESH,
    )
    rdma.start()
    rdma.wait()


def _softmax_body(la_ref, lb_ref, out_ref):
    my_x = lax.axis_index("x")
    a = la_ref[:, :]
    b = lb_ref[:, :]
    m = jnp.maximum(
        jnp.max(a, axis=-1, keepdims=True), jnp.max(b, axis=-1, keepdims=True)
    )
    ea = jnp.exp(a - m)
    eb = jnp.exp(b - m)
    denom = jnp.sum(ea, axis=-1, keepdims=True) + jnp.sum(eb, axis=-1, keepdims=True)
    pa = ea / denom
    pb = eb / denom

    @pl.when(my_x == 0)
    def _():
        out_ref[:, :V_HALF] = pa
        out_ref[:, V_HALF:] = pb

    @pl.when(my_x == 1)
    def _():
        out_ref[:, :V_HALF] = pb
        out_ref[:, V_HALF:] = pa


def kernel(x, W):
    l_local = jnp.dot(x, W, preferred_element_type=jnp.float32)

    l_remote = pl.pallas_call(
        _exchange_body,
        out_shape=jax.ShapeDtypeStruct((T, V_HALF), jnp.float32),
        in_specs=[pl.BlockSpec(memory_space=pltpu.ANY)],
        out_specs=pl.BlockSpec(memory_space=pltpu.ANY),
        scratch_shapes=[
            pltpu.SemaphoreType.DMA,
            pltpu.SemaphoreType.DMA,
        ],
        compiler_params=pltpu.CompilerParams(collective_id=0),
    )(l_local)

    return pl.pallas_call(
        _softmax_body,
        grid=(T // BM,),
        out_shape=jax.ShapeDtypeStruct((T, V), jnp.float32),
        in_specs=[
            pl.BlockSpec((BM, V_HALF), lambda i: (i, 0)),
            pl.BlockSpec((BM, V_HALF), lambda i: (i, 0)),
        ],
        out_specs=pl.BlockSpec((BM, V), lambda i: (i, 0)),
    )(l_local, l_remote)


# baseline (device time: 898907 ns/iter reference)
import jax
import jax.numpy as jnp
from jax import lax
from jax.experimental import pallas as pl
from jax.experimental.pallas import tpu as pltpu

T = 1024
V = 32768
V_HALF = V // 2
BM = 32


def _exchange_body(l_ref, out_ref, send_sem, recv_sem):
    my_x = lax.axis_index("x")
    my_y = lax.axis_index("y")
    my_z = lax.axis_index("z")
    nbr = (1 - my_x, my_y, my_z)

    barrier_sem = pltpu.get_barrier_semaphore()
    pl.semaphore_signal(
        barrier_sem, inc=1, device_id=nbr, device_id_type=pl.DeviceIdType.MESH
    )
    pl.semaphore_wait(barrier_sem, 1)

    rdma = pltpu.make_async_remote_copy(
        src_ref=l_ref,
        dst_ref=out_ref,
        send_sem=send_sem,
        recv_sem=recv_sem,
        device_id=nbr,
        device_id_type=pl.DeviceIdType.MESH,
    )
    rdma.start()
    rdma.wait()


def _softmax_body(la_ref, lb_ref, out_ref):
    my_x = lax.axis_index("x")
    a = la_ref[:, :]
    b = lb_ref[:, :]
    m = jnp.maximum(
        jnp.max(a, axis=-1, keepdims=True), jnp.max(b, axis=-1, keepdims=True)
    )
    ea = jnp.exp(a - m)
    eb = jnp.exp(b - m)
    denom = jnp.sum(ea, axis=-1, keepdims=True) + jnp.sum(eb, axis=-1, keepdims=True)
    pa = ea / denom
    pb = eb / denom

    @pl.when(my_x == 0)
    def _():
        out_ref[:, :V_HALF] = pa
        out_ref[:, V_HALF:] = pb

    @pl.when(my_x == 1)
    def _():
        out_ref[:, :V_HALF] = pb
        out_ref[:, V_HALF:] = pa


def kernel(x, W):
    l_local = jnp.dot(x, W, preferred_element_type=jnp.float32)

    l_remote = pl.pallas_call(
        _exchange_body,
        out_shape=jax.ShapeDtypeStruct((T, V_HALF), jnp.float32),
        in_specs=[pl.BlockSpec(memory_space=pl.ANY)],
        out_specs=pl.BlockSpec(memory_space=pl.ANY),
        scratch_shapes=[
            pltpu.SemaphoreType.DMA,
            pltpu.SemaphoreType.DMA,
        ],
        compiler_params=pltpu.CompilerParams(collective_id=0),
    )(l_local)

    return pl.pallas_call(
        _softmax_body,
        grid=(T // BM,),
        out_shape=jax.ShapeDtypeStruct((T, V), jnp.float32),
        in_specs=[
            pl.BlockSpec((BM, V_HALF), lambda i: (i, 0)),
            pl.BlockSpec((BM, V_HALF), lambda i: (i, 0)),
        ],
        out_specs=pl.BlockSpec((BM, V), lambda i: (i, 0)),
    )(l_local, l_remote)
